# baseline (speedup 1.0000x reference)
import functools

import jax
import jax.numpy as jnp
from jax import lax
from jax.experimental import pallas as pl
from jax.experimental.pallas import tpu as pltpu
from jax.experimental.pallas import tpu_sc as plsc

_N = 65536
_L = 256
_LAT = 32
_TH = 0.15
_NC, _NS, _LANES = 2, 16, 16
_NW = _NC * _NS
_ROWS_PER_W = _N // _NW
_BLK = 128
_NBLK = _ROWS_PER_W // _BLK
_NG = _BLK // _LANES
_SCAN0 = 16
_OP = 2 * _LAT + 1


def _compute_block(x_v, o_tp, o_t, iota16):
    def grp_body(g, carry):
        rowbase = g * _LANES
        fs = []
        for k in range(_LANES):
            head = x_v[rowbase + k, pl.ds(0, _SCAN0)]
            fs.append(plsc.all_reduce_ffs(head > _TH))
        p = jnp.full((_LANES,), _SCAN0, jnp.int32)
        for k in range(_LANES):
            p = jnp.where(iota16 == k, fs[k], p)
        fillv = p.astype(jnp.float32) * (1.0 / _L)
        for j in range(_LAT):
            o_t[_LAT + j, pl.ds(rowbase, _LANES)] = fillv

        for k in range(_LANES):
            rs = jnp.full((_LANES,), rowbase + k, jnp.int32)
            idx1 = fs[k] + iota16
            g1 = plsc.load_gather(x_v, [rs, idx1])
            g2 = plsc.load_gather(x_v, [rs, idx1 + _SCAN0])
            plsc.store_scatter(o_tp, [iota16, rs], g1)
            plsc.store_scatter(o_tp, [iota16 + _SCAN0, rs], g2)

        @pl.when(jnp.any(p >= _SCAN0))
        def _():
            pv = p
            for k in range(_LANES):
                r = rowbase + k

                def chunk(c, rc, r=r):
                    v = x_v[r, pl.ds(c * _SCAN0, _SCAN0)]
                    fc = plsc.all_reduce_ffs(v > _TH)
                    return jnp.minimum(
                        rc, jnp.where(fc < _SCAN0, c * _SCAN0 + fc, 1024))

                rc = lax.fori_loop(
                    1, _L // _SCAN0, chunk, jnp.full((_LANES,), 1024, jnp.int32))
                pv = jnp.where(
                    jnp.logical_and(iota16 == k, pv >= _SCAN0), rc, pv)
            pv = jnp.where(pv >= _L, 0, pv)
            for k in range(_LANES):
                r = rowbase + k
                ps = jnp.full((_LANES,), pv[k], jnp.int32)
                rs = jnp.full((_LANES,), r, jnp.int32)
                idx1 = jnp.bitwise_and(ps + iota16, _L - 1)
                idx2 = jnp.bitwise_and(idx1 + _SCAN0, _L - 1)
                g1 = plsc.load_gather(x_v, [rs, idx1])
                g2 = plsc.load_gather(x_v, [rs, idx2])
                plsc.store_scatter(o_tp, [iota16, rs], g1)
                plsc.store_scatter(o_tp, [iota16 + _SCAN0, rs], g2)
            fillp = pv.astype(jnp.float32) * (1.0 / _L)
            for j in range(_LAT):
                o_t[_LAT + j, pl.ds(rowbase, _LANES)] = fillp

        return 0

    lax.fori_loop(0, _NG, grp_body, 0)

    @plsc.parallel_loop(0, _LAT, unroll=2)
    def _(j):
        for q in range(_NG):
            o_t[j, pl.ds(q * _LANES, _LANES)] = o_tp[j, pl.ds(q * _LANES, _LANES)]


def _sc_body(x_hbm, out_hbm, x0, x1, ot0, ot1, o_pad, si0, si1, so0, so1):
    wid = lax.axis_index("s") * _NC + lax.axis_index("c")
    base = wid * _ROWS_PER_W
    iota = lax.iota(jnp.int32, _LANES)

    def in_copy(b, buf, sem):
        return pltpu.make_async_copy(
            x_hbm.at[pl.ds(base + b * _BLK, _BLK)], buf, sem)

    def out_copy(b, buf, sem):
        return pltpu.make_async_copy(
            buf, out_hbm.at[:, pl.ds(base + b * _BLK, _BLK)], sem)

    in_copy(0, x0, si0).start()

    def pair_body(t, carry):
        b0 = 2 * t
        in_copy(b0 + 1, x1, si1).start()
        in_copy(b0, x0, si0).wait()

        @pl.when(t > 0)
        def _():
            out_copy(2 * t - 2, ot0, so0).wait()

        _compute_block(x0, o_pad, ot0, iota)
        out_copy(b0, ot0, so0).start()

        @pl.when(t < _NBLK // 2 - 1)
        def _():
            in_copy(b0 + 2, x0, si0).start()
        in_copy(b0 + 1, x1, si1).wait()

        @pl.when(t > 0)
        def _():
            out_copy(2 * t - 1, ot1, so1).wait()

        _compute_block(x1, o_pad, ot1, iota)
        out_copy(b0 + 1, ot1, so1).start()
        return carry

    lax.fori_loop(0, _NBLK // 2, pair_body, 0)
    out_copy(_NBLK - 2, ot0, so0).wait()
    out_copy(_NBLK - 1, ot1, so1).wait()


@jax.jit
def kernel(x):
    n = x.shape[0]
    mesh = plsc.VectorSubcoreMesh(core_axis_name="c", subcore_axis_name="s")
    run = functools.partial(
        pl.kernel,
        out_type=jax.ShapeDtypeStruct((2 * _LAT, n), jnp.float32),
        mesh=mesh,
        scratch_types=[
            pltpu.VMEM((_BLK, _L), jnp.float32),
            pltpu.VMEM((_BLK, _L), jnp.float32),
            pltpu.VMEM((2 * _LAT, _BLK), jnp.float32),
            pltpu.VMEM((2 * _LAT, _BLK), jnp.float32),
            pltpu.VMEM((_LAT, _BLK + 8), jnp.float32),
            pltpu.SemaphoreType.DMA,
            pltpu.SemaphoreType.DMA,
            pltpu.SemaphoreType.DMA,
            pltpu.SemaphoreType.DMA,
        ],
        compiler_params=pltpu.CompilerParams(needs_layout_passes=False),
    )(_sc_body)
    out_t = run(x)
    return jnp.transpose(out_t, (1, 0)).reshape(n, 1, 2 * _LAT)


# speedup vs baseline: 1.0265x; 1.0265x over previous
import functools

import jax
import jax.numpy as jnp
from jax import lax
from jax.experimental import pallas as pl
from jax.experimental.pallas import tpu as pltpu
from jax.experimental.pallas import tpu_sc as plsc

_N = 65536
_L = 256
_LAT = 32
_TH = 0.15
_NC, _NS, _LANES = 2, 16, 16
_NW = _NC * _NS
_ROWS_PER_W = _N // _NW
_BLK = 128
_NBLK = _ROWS_PER_W // _BLK
_NG = _BLK // _LANES
_SCAN0 = 16
_OP = 2 * _LAT + 1


def _compute_block(x_v, o_tp, o_t, iota16):
    def grp_body(g, carry):
        rowbase = g * _LANES
        fs = []
        for k in range(_LANES):
            head = x_v[rowbase + k, pl.ds(0, _SCAN0)]
            fs.append(plsc.all_reduce_ffs(head > _TH))
        p = jnp.full((_LANES,), _SCAN0, jnp.int32)
        for k in range(_LANES):
            p = jnp.where(iota16 == k, fs[k], p)
        fillv = p.astype(jnp.float32) * (1.0 / _L)
        for j in range(_LAT):
            o_t[_LAT + j, pl.ds(rowbase, _LANES)] = fillv

        for k in range(_LANES):
            rs = jnp.full((_LANES,), rowbase + k, jnp.int32)
            idx1 = fs[k] + iota16
            g1 = plsc.load_gather(x_v, [rs, idx1])
            g2 = plsc.load_gather(x_v, [rs, idx1 + _SCAN0])
            plsc.store_scatter(o_t, [iota16, rs], g1)
            plsc.store_scatter(o_t, [iota16 + _SCAN0, rs], g2)

        @pl.when(jnp.any(p >= _SCAN0))
        def _():
            pv = p
            for k in range(_LANES):
                r = rowbase + k

                def chunk(c, rc, r=r):
                    v = x_v[r, pl.ds(c * _SCAN0, _SCAN0)]
                    fc = plsc.all_reduce_ffs(v > _TH)
                    return jnp.minimum(
                        rc, jnp.where(fc < _SCAN0, c * _SCAN0 + fc, 1024))

                rc = lax.fori_loop(
                    1, _L // _SCAN0, chunk, jnp.full((_LANES,), 1024, jnp.int32))
                pv = jnp.where(
                    jnp.logical_and(iota16 == k, pv >= _SCAN0), rc, pv)
            pv = jnp.where(pv >= _L, 0, pv)
            for k in range(_LANES):
                r = rowbase + k
                ps = jnp.full((_LANES,), pv[k], jnp.int32)
                rs = jnp.full((_LANES,), r, jnp.int32)
                idx1 = jnp.bitwise_and(ps + iota16, _L - 1)
                idx2 = jnp.bitwise_and(idx1 + _SCAN0, _L - 1)
                g1 = plsc.load_gather(x_v, [rs, idx1])
                g2 = plsc.load_gather(x_v, [rs, idx2])
                plsc.store_scatter(o_t, [iota16, rs], g1)
                plsc.store_scatter(o_t, [iota16 + _SCAN0, rs], g2)
            fillp = pv.astype(jnp.float32) * (1.0 / _L)
            for j in range(_LAT):
                o_t[_LAT + j, pl.ds(rowbase, _LANES)] = fillp

        return 0

    lax.fori_loop(0, _NG, grp_body, 0)


def _sc_body(x_hbm, out_hbm, x0, x1, ot0, ot1, o_pad, si0, si1, so0, so1):
    wid = lax.axis_index("s") * _NC + lax.axis_index("c")
    base = wid * _ROWS_PER_W
    iota = lax.iota(jnp.int32, _LANES)

    def in_copy(b, buf, sem):
        return pltpu.make_async_copy(
            x_hbm.at[pl.ds(base + b * _BLK, _BLK)], buf, sem)

    def out_copy(b, buf, sem):
        return pltpu.make_async_copy(
            buf, out_hbm.at[:, pl.ds(base + b * _BLK, _BLK)], sem)

    in_copy(0, x0, si0).start()

    def pair_body(t, carry):
        b0 = 2 * t
        in_copy(b0 + 1, x1, si1).start()
        in_copy(b0, x0, si0).wait()

        @pl.when(t > 0)
        def _():
            out_copy(2 * t - 2, ot0, so0).wait()

        _compute_block(x0, o_pad, ot0, iota)
        out_copy(b0, ot0, so0).start()

        @pl.when(t < _NBLK // 2 - 1)
        def _():
            in_copy(b0 + 2, x0, si0).start()
        in_copy(b0 + 1, x1, si1).wait()

        @pl.when(t > 0)
        def _():
            out_copy(2 * t - 1, ot1, so1).wait()

        _compute_block(x1, o_pad, ot1, iota)
        out_copy(b0 + 1, ot1, so1).start()
        return carry

    lax.fori_loop(0, _NBLK // 2, pair_body, 0)
    out_copy(_NBLK - 2, ot0, so0).wait()
    out_copy(_NBLK - 1, ot1, so1).wait()


@jax.jit
def kernel(x):
    n = x.shape[0]
    mesh = plsc.VectorSubcoreMesh(core_axis_name="c", subcore_axis_name="s")
    run = functools.partial(
        pl.kernel,
        out_type=jax.ShapeDtypeStruct((2 * _LAT, n), jnp.float32),
        mesh=mesh,
        scratch_types=[
            pltpu.VMEM((_BLK, _L), jnp.float32),
            pltpu.VMEM((_BLK, _L), jnp.float32),
            pltpu.VMEM((2 * _LAT, _BLK), jnp.float32),
            pltpu.VMEM((2 * _LAT, _BLK), jnp.float32),
            pltpu.VMEM((_LAT, _BLK + 8), jnp.float32),
            pltpu.SemaphoreType.DMA,
            pltpu.SemaphoreType.DMA,
            pltpu.SemaphoreType.DMA,
            pltpu.SemaphoreType.DMA,
        ],
        compiler_params=pltpu.CompilerParams(needs_layout_passes=False),
    )(_sc_body)
    out_t = run(x)
    return jnp.transpose(out_t, (1, 0)).reshape(n, 1, 2 * _LAT)
